# baseline (speedup 1.0000x reference)
import functools

import jax
import jax.numpy as jnp
from jax import lax
from jax.experimental import pallas as pl
from jax.experimental.pallas import tpu as pltpu
from jax.experimental.pallas import tpu_sc as plsc

_B, _NP, _DIM, _K = 4, 1024, 128, 16
_HID, _NPROP = 64, 5
_NBLK = 256
_HI = jax.lax.Precision.HIGHEST


def _sigmoid(u):
    return 1.0 / (1.0 + jnp.exp(-u))


def _softplus(v):
    return jnp.maximum(v, 0.0) + jnp.log1p(jnp.exp(-jnp.abs(v)))


def _knn_cgconv_body(xyz_ref, xyzt_ref, x_ref, wf_ref, bf_ref, ws_ref, bs_ref,
                     agg_ref, sums_ref, p_ref, r_ref, qsh_ref, qsl_ref):
    g = pl.program_id(0)
    b_in_graph = jax.lax.rem(g, _B)

    x = x_ref[0].astype(jnp.bfloat16)
    wf = wf_ref[...].astype(jnp.bfloat16)
    ws = ws_ref[...].astype(jnp.bfloat16)
    dot = functools.partial(jnp.dot, preferred_element_type=jnp.float32)
    p_ref[...] = dot(x, wf[:_DIM]) + bf_ref[...]
    r_ref[...] = dot(x, ws[:_DIM]) + bs_ref[...]
    q = dot(x, wf[_DIM:])
    s = dot(x, ws[_DIM:])
    qs = jnp.concatenate([q, s], axis=1)
    hi = qs.astype(jnp.bfloat16)
    qsh_ref[...] = hi
    qsl_ref[...] = (qs - hi.astype(jnp.float32)).astype(jnp.bfloat16)

    total_sum = jnp.zeros((1, _DIM), jnp.float32)
    total_sq = jnp.zeros((1, _DIM), jnp.float32)

    for nb in range(_NP // _NBLK):
        base = nb * _NBLK
        lane = jax.lax.broadcasted_iota(jnp.int32, (_NBLK, _NP), 1)
        rows = base + jax.lax.broadcasted_iota(jnp.int32, (_NBLK, _NP), 0)
        d2 = jnp.zeros((_NBLK, _NP), jnp.float32)
        for c in range(3):
            col = xyz_ref[0, pl.ds(base, _NBLK), c:c + 1]
            row = xyzt_ref[0, c:c + 1, :]
            t = col - row
            d2 = d2 + t * t
        d2 = d2 + jnp.where(lane == rows, 1e10, 0.0)

        p_blk = p_ref[pl.ds(base, _NBLK), :]
        r_blk = r_ref[pl.ds(base, _NBLK), :]
        qsh = qsh_ref[...]
        qsl = qsl_ref[...]
        acc = None
        for k in range(_K):
            m = jnp.min(d2, axis=1, keepdims=True)
            cur = jnp.min(jnp.where(d2 == m, lane, jnp.int32(2**30)),
                          axis=1, keepdims=True)
            hit = lane == cur
            d2 = jnp.where(hit, 3e30, d2)
            oh = jnp.where(hit, 1.0, 0.0).astype(jnp.bfloat16)
            gh = jax.lax.dot_general(oh, qsh, (((1,), (0,)), ((), ())),
                                     preferred_element_type=jnp.float32)
            gl = jax.lax.dot_general(oh, qsl, (((1,), (0,)), ((), ())),
                                     preferred_element_type=jnp.float32)
            gat = gh + gl
            u = p_blk + gat[:, :_DIM]
            v = r_blk + gat[:, _DIM:]
            msg = _sigmoid(u) * _softplus(v)
            acc = msg if acc is None else jnp.maximum(acc, msg)
        agg_ref[0, pl.ds(base, _NBLK), :] = acc
        total_sum = total_sum + jnp.sum(acc, axis=0, keepdims=True)
        total_sq = total_sq + jnp.sum(acc * acc, axis=0, keepdims=True)

    local = jnp.concatenate([total_sum[:, None, :], total_sq[:, None, :]],
                            axis=1)

    @pl.when(b_in_graph == 0)
    def _():
        sums_ref[...] = local

    @pl.when(b_in_graph != 0)
    def _():
        sums_ref[...] = sums_ref[...] + local


def _knn_pqrs_body(xyz_ref, xyzt_ref, x_ref, wf_ref, bf_ref, ws_ref, bs_ref,
                   p_ref, r_ref, qs_ref, idx_ref):
    x = x_ref[0].astype(jnp.bfloat16)
    wf = wf_ref[...].astype(jnp.bfloat16)
    ws = ws_ref[...].astype(jnp.bfloat16)
    dot = functools.partial(jnp.dot, preferred_element_type=jnp.float32)
    p_ref[0] = dot(x, wf[:_DIM]) + bf_ref[...]
    r_ref[0] = dot(x, ws[:_DIM]) + bs_ref[...]
    qs_ref[0, :, :_DIM] = dot(x, wf[_DIM:])
    qs_ref[0, :, _DIM:] = dot(x, ws[_DIM:])

    for nb in range(_NP // _NBLK):
        base = nb * _NBLK
        lane = jax.lax.broadcasted_iota(jnp.int32, (_NBLK, _NP), 1)
        rows = base + jax.lax.broadcasted_iota(jnp.int32, (_NBLK, _NP), 0)
        d2 = jnp.zeros((_NBLK, _NP), jnp.float32)
        for c in range(3):
            col = xyz_ref[0, pl.ds(base, _NBLK), c:c + 1]
            row = xyzt_ref[0, c:c + 1, :]
            t = col - row
            d2 = d2 + t * t
        d2 = d2 + jnp.where(lane == rows, 1e10, 0.0)
        for k in range(_K):
            m = jnp.min(d2, axis=1, keepdims=True)
            cur = jnp.min(jnp.where(d2 == m, lane, jnp.int32(2**30)),
                          axis=1, keepdims=True)
            d2 = jnp.where(lane == cur, 3e30, d2)
            idx_ref[0, pl.ds(base, _NBLK), k:k + 1] = cur


_NW = 32
_ECH = 128


def _sc_gather_body(table_ref, idx_ref, out_ref, idx_v, rows_v, sem):
    wid = lax.axis_index("s") * 2 + lax.axis_index("c")
    per_w = (2 * _B * _NP * _K) // _NW
    base = wid * per_w

    @pl.loop(0, per_w // _ECH)
    def _(j):
        off = base + j * _ECH
        pltpu.sync_copy(idx_ref.at[pl.ds(off, _ECH)], idx_v)
        pltpu.async_copy(table_ref.at[idx_v], rows_v, sem).wait()
        pltpu.sync_copy(rows_v, out_ref.at[pl.ds(off, _ECH)])


def _msg_max_body(g_ref, p_ref, r_ref, agg_ref, sums_ref):
    g = pl.program_id(0)
    nb = pl.program_id(1)
    p = p_ref[...]
    r = r_ref[...]
    acc = None
    for k in range(_K):
        gat = g_ref[k]
        u = p + gat[:, :_DIM]
        v = r + gat[:, _DIM:]
        msg = _sigmoid(u) * _softplus(v)
        acc = msg if acc is None else jnp.maximum(acc, msg)
    agg_ref[...] = acc
    bsum = jnp.sum(acc, axis=0, keepdims=True)
    bsq = jnp.sum(acc * acc, axis=0, keepdims=True)
    local = jnp.concatenate([bsum[:, None, :], bsq[:, None, :]], axis=1)

    first = jnp.logical_and(lax.rem(g, _B) == 0, nb == 0)

    @pl.when(first)
    def _():
        sums_ref[...] = local

    @pl.when(jnp.logical_not(first))
    def _():
        sums_ref[...] = sums_ref[...] + local


def _cross_prop_body(x0_ref, x1_ref, agg0_ref, agg1_ref, sums_ref,
                     gamma_ref, beta_ref, wh_ref, bh_ref, wo_ref, bo_ref,
                     out0_ref, out1_ref, f0_ref, f1_ref, h0_ref, h1_ref):
    gamma = gamma_ref[...]
    beta = beta_ref[...]
    n_nodes = jnp.float32(_B * _NP)

    for gi, (x_ref, agg_ref, f_ref) in enumerate(
            ((x0_ref, agg0_ref, f0_ref), (x1_ref, agg1_ref, f1_ref))):
        ssum = sums_ref[gi, 0:1, :]
        ssq = sums_ref[gi, 1:2, :]
        mu = ssum / n_nodes
        var = ssq / n_nodes - mu * mu
        inv = jax.lax.rsqrt(var + 1e-5)
        f_ref[...] = x_ref[0] + (agg_ref[0] - mu) * inv * gamma + beta

    def rowsoftmax(t):
        m = jnp.max(t, axis=1, keepdims=True)
        e = jnp.exp(t - m)
        return e / jnp.sum(e, axis=1, keepdims=True)

    bf16 = jnp.bfloat16
    dot = functools.partial(jnp.dot, preferred_element_type=jnp.float32)
    for l in range(_NPROP):
        wh = wh_ref[l].astype(bf16)
        bh = bh_ref[l]
        wo = wo_ref[l].astype(bf16)
        bo = bo_ref[l]
        f0 = f0_ref[...].astype(bf16)
        f1 = f1_ref[...].astype(bf16)
        for nb in range(_NP // _NBLK):
            base = nb * _NBLK
            f0b = f0[base:base + _NBLK, :]
            f1b = f1[base:base + _NBLK, :]
            s = jax.lax.dot_general(f0b, f1, (((1,), (1,)), ((), ())),
                                    preferred_element_type=jnp.float32)
            st = jax.lax.dot_general(f1b, f0, (((1,), (1,)), ((), ())),
                                     preferred_element_type=jnp.float32)
            a0 = rowsoftmax(s).astype(bf16)
            a1t = rowsoftmax(st).astype(bf16)
            att0 = dot(a0, f1)
            att1 = dot(a1t, f0)
            mu0 = f0_ref[pl.ds(base, _NBLK), :] - att0
            mu1 = f1_ref[pl.ds(base, _NBLK), :] - att1
            h0 = dot(jax.nn.relu(
                dot(f0b, wh[:_DIM])
                + dot(mu0.astype(bf16), wh[_DIM:]) + bh).astype(bf16),
                wo) + bo
            h1 = dot(jax.nn.relu(
                dot(f1b, wh[:_DIM])
                + dot(mu1.astype(bf16), wh[_DIM:]) + bh).astype(bf16),
                wo) + bo
            h0_ref[pl.ds(base, _NBLK), :] = h0
            h1_ref[pl.ds(base, _NBLK), :] = h1
        f0_ref[...] = f0_ref[...] + h0_ref[...]
        f1_ref[...] = f1_ref[...] + h1_ref[...]

    out0_ref[0] = f0_ref[...]
    out1_ref[0] = f1_ref[...]


bcast = lambda shape: pl.BlockSpec(shape, lambda g: (0,) * len(shape))
per_g = lambda shape: pl.BlockSpec(shape, lambda g: (g,) + (0,) * (len(shape) - 1))


def _kernel_tc(input_xyz, coord_xyz, input_f, coord_f, Wf, bf, Ws, bs, gamma,
               beta, Wh, bh, Wo, bo):
    f32 = jnp.float32
    xyz = jnp.concatenate([input_xyz, coord_xyz], axis=0)
    xyzt = jnp.pad(jnp.transpose(xyz, (0, 2, 1)),
                   ((0, 0), (0, 5), (0, 0)))
    x_s = jnp.concatenate([input_f, coord_f], axis=0)

    agg, sums = pl.pallas_call(
        _knn_cgconv_body,
        grid=(2 * _B,),
        in_specs=[
            per_g((1, _NP, 3)),
            per_g((1, 8, _NP)),
            per_g((1, _NP, _DIM)),
            bcast((2 * _DIM, _DIM)),
            bcast((1, _DIM)),
            bcast((2 * _DIM, _DIM)),
            bcast((1, _DIM)),
        ],
        out_specs=[
            per_g((1, _NP, _DIM)),
            pl.BlockSpec((1, 2, _DIM), lambda g: (g // _B, 0, 0)),
        ],
        out_shape=[
            jax.ShapeDtypeStruct((2 * _B, _NP, _DIM), f32),
            jax.ShapeDtypeStruct((2, 2, _DIM), f32),
        ],
        scratch_shapes=[
            pltpu.VMEM((_NP, _DIM), f32),
            pltpu.VMEM((_NP, _DIM), f32),
            pltpu.VMEM((_NP, 2 * _DIM), jnp.bfloat16),
            pltpu.VMEM((_NP, 2 * _DIM), jnp.bfloat16),
        ],
    )(xyz, xyzt, x_s, Wf, bf.reshape(1, _DIM), Ws, bs.reshape(1, _DIM))
    return _run_cross_prop(agg, sums, input_f, coord_f, gamma, beta,
                           Wh, bh, Wo, bo)


def _run_cross_prop(agg, sums, input_f, coord_f, gamma, beta, Wh, bh, Wo, bo):
    f32 = jnp.float32
    out0, out1 = pl.pallas_call(
        _cross_prop_body,
        grid=(_B,),
        in_specs=[
            per_g((1, _NP, _DIM)),
            per_g((1, _NP, _DIM)),
            pl.BlockSpec((1, _NP, _DIM), lambda b: (b, 0, 0)),
            pl.BlockSpec((1, _NP, _DIM), lambda b: (b + _B, 0, 0)),
            bcast((2, 2, _DIM)),
            bcast((1, _DIM)),
            bcast((1, _DIM)),
            bcast((_NPROP, 2 * _DIM, _HID)),
            bcast((_NPROP, 1, _HID)),
            bcast((_NPROP, _HID, _DIM)),
            bcast((_NPROP, 1, _DIM)),
        ],
        out_specs=[
            per_g((1, _NP, _DIM)),
            per_g((1, _NP, _DIM)),
        ],
        out_shape=[
            jax.ShapeDtypeStruct((_B, _NP, _DIM), f32),
            jax.ShapeDtypeStruct((_B, _NP, _DIM), f32),
        ],
        scratch_shapes=[
            pltpu.VMEM((_NP, _DIM), f32),
            pltpu.VMEM((_NP, _DIM), f32),
            pltpu.VMEM((_NP, _DIM), f32),
            pltpu.VMEM((_NP, _DIM), f32),
        ],
    )(input_f, coord_f, agg, agg, sums, gamma.reshape(1, _DIM),
      beta.reshape(1, _DIM), Wh, bh.reshape(_NPROP, 1, _HID), Wo,
      bo.reshape(_NPROP, 1, _DIM))

    return (out0.reshape(-1, _DIM), out1.reshape(-1, _DIM))


def _kernel_sc(input_xyz, coord_xyz, input_f, coord_f, Wf, bf, Ws, bs, gamma,
               beta, Wh, bh, Wo, bo):
    f32 = jnp.float32
    n_nodes = 2 * _B * _NP
    n_edges = n_nodes * _K
    xyz = jnp.concatenate([input_xyz, coord_xyz], axis=0)
    xyzt = jnp.pad(jnp.transpose(xyz, (0, 2, 1)),
                   ((0, 0), (0, 5), (0, 0)))
    x_s = jnp.concatenate([input_f, coord_f], axis=0)

    p, r, qs, idx = pl.pallas_call(
        _knn_pqrs_body,
        grid=(2 * _B,),
        in_specs=[
            per_g((1, _NP, 3)),
            per_g((1, 8, _NP)),
            per_g((1, _NP, _DIM)),
            bcast((2 * _DIM, _DIM)),
            bcast((1, _DIM)),
            bcast((2 * _DIM, _DIM)),
            bcast((1, _DIM)),
        ],
        out_specs=[
            per_g((1, _NP, _DIM)),
            per_g((1, _NP, _DIM)),
            per_g((1, _NP, 2 * _DIM)),
            per_g((1, _NP, _K)),
        ],
        out_shape=[
            jax.ShapeDtypeStruct((2 * _B, _NP, _DIM), f32),
            jax.ShapeDtypeStruct((2 * _B, _NP, _DIM), f32),
            jax.ShapeDtypeStruct((2 * _B, _NP, 2 * _DIM), f32),
            jax.ShapeDtypeStruct((2 * _B, _NP, _K), jnp.int32),
        ],
    )(xyz, xyzt, x_s, Wf, bf.reshape(1, _DIM), Ws, bs.reshape(1, _DIM))

    gidx = idx + (jnp.arange(2 * _B, dtype=jnp.int32) * _NP)[:, None, None]
    eidx = jnp.transpose(gidx.reshape(n_nodes, _K)).reshape(-1)

    mesh = plsc.VectorSubcoreMesh(core_axis_name="c", subcore_axis_name="s")
    gathered = pl.kernel(
        _sc_gather_body,
        mesh=mesh,
        out_type=jax.ShapeDtypeStruct((n_edges, 2 * _DIM), f32),
        scratch_types=[
            pltpu.VMEM((_ECH,), jnp.int32),
            pltpu.VMEM((_ECH, 2 * _DIM), f32),
            pltpu.SemaphoreType.DMA,
        ],
    )(qs.reshape(n_nodes, 2 * _DIM), eidx)

    agg, sums = pl.pallas_call(
        _msg_max_body,
        grid=(2 * _B, _NP // _NBLK),
        in_specs=[
            pl.BlockSpec((_K, _NBLK, 2 * _DIM),
                         lambda g, nb: (0, g * (_NP // _NBLK) + nb, 0)),
            pl.BlockSpec((_NBLK, _DIM),
                         lambda g, nb: (g * (_NP // _NBLK) + nb, 0)),
            pl.BlockSpec((_NBLK, _DIM),
                         lambda g, nb: (g * (_NP // _NBLK) + nb, 0)),
        ],
        out_specs=[
            pl.BlockSpec((_NBLK, _DIM),
                         lambda g, nb: (g * (_NP // _NBLK) + nb, 0)),
            pl.BlockSpec((1, 2, _DIM), lambda g, nb: (g // _B, 0, 0)),
        ],
        out_shape=[
            jax.ShapeDtypeStruct((n_nodes, _DIM), f32),
            jax.ShapeDtypeStruct((2, 2, _DIM), f32),
        ],
    )(gathered.reshape(_K, n_nodes, 2 * _DIM),
      p.reshape(n_nodes, _DIM), r.reshape(n_nodes, _DIM))

    return _run_cross_prop(agg.reshape(2 * _B, _NP, _DIM), sums, input_f,
                           coord_f, gamma, beta, Wh, bh, Wo, bo)


def kernel(input_xyz, coord_xyz, input_f, coord_f, Wf, bf, Ws, bs, gamma,
           beta, Wh, bh, Wo, bo):
    return _kernel_sc(input_xyz, coord_xyz, input_f, coord_f, Wf, bf, Ws,
                      bs, gamma, beta, Wh, bh, Wo, bo)


# speedup vs baseline: 1.0219x; 1.0219x over previous
import functools

import jax
import jax.numpy as jnp
from jax import lax
from jax.experimental import pallas as pl
from jax.experimental.pallas import tpu as pltpu
from jax.experimental.pallas import tpu_sc as plsc

_B, _NP, _DIM, _K = 4, 1024, 128, 16
_HID, _NPROP = 64, 5
_NBLK = 256
_HI = jax.lax.Precision.HIGHEST


def _sigmoid(u):
    return 1.0 / (1.0 + jnp.exp(-u))


def _softplus(v):
    return jnp.maximum(v, 0.0) + jnp.log1p(jnp.exp(-jnp.abs(v)))


def _knn_cgconv_body(xyz_ref, xyzt_ref, x_ref, wf_ref, bf_ref, ws_ref, bs_ref,
                     agg_ref, sums_ref, p_ref, r_ref, qsh_ref, qsl_ref):
    g = pl.program_id(0)
    b_in_graph = jax.lax.rem(g, _B)

    x = x_ref[0].astype(jnp.bfloat16)
    wf = wf_ref[...].astype(jnp.bfloat16)
    ws = ws_ref[...].astype(jnp.bfloat16)
    dot = functools.partial(jnp.dot, preferred_element_type=jnp.float32)
    p_ref[...] = dot(x, wf[:_DIM]) + bf_ref[...]
    r_ref[...] = dot(x, ws[:_DIM]) + bs_ref[...]
    q = dot(x, wf[_DIM:])
    s = dot(x, ws[_DIM:])
    qs = jnp.concatenate([q, s], axis=1)
    hi = qs.astype(jnp.bfloat16)
    qsh_ref[...] = hi
    qsl_ref[...] = (qs - hi.astype(jnp.float32)).astype(jnp.bfloat16)

    total_sum = jnp.zeros((1, _DIM), jnp.float32)
    total_sq = jnp.zeros((1, _DIM), jnp.float32)

    for nb in range(_NP // _NBLK):
        base = nb * _NBLK
        lane = jax.lax.broadcasted_iota(jnp.int32, (_NBLK, _NP), 1)
        rows = base + jax.lax.broadcasted_iota(jnp.int32, (_NBLK, _NP), 0)
        d2 = jnp.zeros((_NBLK, _NP), jnp.float32)
        for c in range(3):
            col = xyz_ref[0, pl.ds(base, _NBLK), c:c + 1]
            row = xyzt_ref[0, c:c + 1, :]
            t = col - row
            d2 = d2 + t * t
        d2 = d2 + jnp.where(lane == rows, 1e10, 0.0)

        p_blk = p_ref[pl.ds(base, _NBLK), :]
        r_blk = r_ref[pl.ds(base, _NBLK), :]
        qsh = qsh_ref[...]
        qsl = qsl_ref[...]
        acc = None
        for k in range(_K):
            m = jnp.min(d2, axis=1, keepdims=True)
            cur = jnp.min(jnp.where(d2 == m, lane, jnp.int32(2**30)),
                          axis=1, keepdims=True)
            hit = lane == cur
            d2 = jnp.where(hit, 3e30, d2)
            oh = jnp.where(hit, 1.0, 0.0).astype(jnp.bfloat16)
            gh = jax.lax.dot_general(oh, qsh, (((1,), (0,)), ((), ())),
                                     preferred_element_type=jnp.float32)
            gl = jax.lax.dot_general(oh, qsl, (((1,), (0,)), ((), ())),
                                     preferred_element_type=jnp.float32)
            gat = gh + gl
            u = p_blk + gat[:, :_DIM]
            v = r_blk + gat[:, _DIM:]
            msg = _sigmoid(u) * _softplus(v)
            acc = msg if acc is None else jnp.maximum(acc, msg)
        agg_ref[0, pl.ds(base, _NBLK), :] = acc
        total_sum = total_sum + jnp.sum(acc, axis=0, keepdims=True)
        total_sq = total_sq + jnp.sum(acc * acc, axis=0, keepdims=True)

    local = jnp.concatenate([total_sum[:, None, :], total_sq[:, None, :]],
                            axis=1)

    @pl.when(b_in_graph == 0)
    def _():
        sums_ref[...] = local

    @pl.when(b_in_graph != 0)
    def _():
        sums_ref[...] = sums_ref[...] + local


def _knn_pqrs_body(xyz_ref, xyzt_ref, x_ref, wf_ref, bf_ref, ws_ref, bs_ref,
                   p_ref, r_ref, qs_ref, idx_ref):
    x = x_ref[0].astype(jnp.bfloat16)
    wf = wf_ref[...].astype(jnp.bfloat16)
    ws = ws_ref[...].astype(jnp.bfloat16)
    dot = functools.partial(jnp.dot, preferred_element_type=jnp.float32)
    p_ref[0] = dot(x, wf[:_DIM]) + bf_ref[...]
    r_ref[0] = dot(x, ws[:_DIM]) + bs_ref[...]
    qs_ref[0, :, :_DIM] = dot(x, wf[_DIM:])
    qs_ref[0, :, _DIM:] = dot(x, ws[_DIM:])

    for nb in range(_NP // _NBLK):
        base = nb * _NBLK
        lane = jax.lax.broadcasted_iota(jnp.int32, (_NBLK, _NP), 1)
        rows = base + jax.lax.broadcasted_iota(jnp.int32, (_NBLK, _NP), 0)
        d2 = jnp.zeros((_NBLK, _NP), jnp.float32)
        for c in range(3):
            col = xyz_ref[0, pl.ds(base, _NBLK), c:c + 1]
            row = xyzt_ref[0, c:c + 1, :]
            t = col - row
            d2 = d2 + t * t
        d2 = d2 + jnp.where(lane == rows, 1e10, 0.0)
        for k in range(_K):
            m = jnp.min(d2, axis=1, keepdims=True)
            cur = jnp.min(jnp.where(d2 == m, lane, jnp.int32(2**30)),
                          axis=1, keepdims=True)
            d2 = jnp.where(lane == cur, 3e30, d2)
            idx_ref[0, pl.ds(base, _NBLK), k:k + 1] = cur


_NW = 32
_ECH = 128


def _sc_gather_body(table_ref, idx_ref, out_ref, idx_v, rows_v, sem):
    wid = lax.axis_index("s") * 2 + lax.axis_index("c")
    per_w = (_B * _NP * _K) // _NW
    base = wid * per_w

    @pl.loop(0, per_w // _ECH)
    def _(j):
        off = base + j * _ECH
        pltpu.sync_copy(idx_ref.at[pl.ds(off, _ECH)], idx_v)
        pltpu.async_copy(table_ref.at[idx_v], rows_v, sem).wait()
        pltpu.sync_copy(rows_v, out_ref.at[pl.ds(off, _ECH)])


def _msg_max_body(g_ref, p_ref, r_ref, agg_ref, sums_ref):
    g = pl.program_id(0)
    nb = pl.program_id(1)
    p = p_ref[...]
    r = r_ref[...]
    acc = None
    for k in range(_K):
        gat = g_ref[k]
        u = p + gat[:, :_DIM]
        v = r + gat[:, _DIM:]
        msg = _sigmoid(u) * _softplus(v)
        acc = msg if acc is None else jnp.maximum(acc, msg)
    agg_ref[...] = acc
    bsum = jnp.sum(acc, axis=0, keepdims=True)
    bsq = jnp.sum(acc * acc, axis=0, keepdims=True)
    local = jnp.concatenate([bsum[:, None, :], bsq[:, None, :]], axis=1)

    first = jnp.logical_and(lax.rem(g, _B) == 0, nb == 0)

    @pl.when(first)
    def _():
        sums_ref[...] = local

    @pl.when(jnp.logical_not(first))
    def _():
        sums_ref[...] = sums_ref[...] + local


def _cross_prop_body(x0_ref, x1_ref, agg0_ref, agg1_ref, sums_ref,
                     gamma_ref, beta_ref, wh_ref, bh_ref, wo_ref, bo_ref,
                     out0_ref, out1_ref, f0_ref, f1_ref, h0_ref, h1_ref):
    gamma = gamma_ref[...]
    beta = beta_ref[...]
    n_nodes = jnp.float32(_B * _NP)

    for gi, (x_ref, agg_ref, f_ref) in enumerate(
            ((x0_ref, agg0_ref, f0_ref), (x1_ref, agg1_ref, f1_ref))):
        ssum = sums_ref[gi, 0:1, :]
        ssq = sums_ref[gi, 1:2, :]
        mu = ssum / n_nodes
        var = ssq / n_nodes - mu * mu
        inv = jax.lax.rsqrt(var + 1e-5)
        f_ref[...] = x_ref[0] + (agg_ref[0] - mu) * inv * gamma + beta

    def rowsoftmax(t):
        m = jnp.max(t, axis=1, keepdims=True)
        e = jnp.exp(t - m)
        return e / jnp.sum(e, axis=1, keepdims=True)

    bf16 = jnp.bfloat16
    dot = functools.partial(jnp.dot, preferred_element_type=jnp.float32)
    for l in range(_NPROP):
        wh = wh_ref[l].astype(bf16)
        bh = bh_ref[l]
        wo = wo_ref[l].astype(bf16)
        bo = bo_ref[l]
        f0 = f0_ref[...].astype(bf16)
        f1 = f1_ref[...].astype(bf16)
        for nb in range(_NP // _NBLK):
            base = nb * _NBLK
            f0b = f0[base:base + _NBLK, :]
            f1b = f1[base:base + _NBLK, :]
            s = jax.lax.dot_general(f0b, f1, (((1,), (1,)), ((), ())),
                                    preferred_element_type=jnp.float32)
            st = jax.lax.dot_general(f1b, f0, (((1,), (1,)), ((), ())),
                                     preferred_element_type=jnp.float32)
            a0 = rowsoftmax(s).astype(bf16)
            a1t = rowsoftmax(st).astype(bf16)
            att0 = dot(a0, f1)
            att1 = dot(a1t, f0)
            mu0 = f0_ref[pl.ds(base, _NBLK), :] - att0
            mu1 = f1_ref[pl.ds(base, _NBLK), :] - att1
            h0 = dot(jax.nn.relu(
                dot(f0b, wh[:_DIM])
                + dot(mu0.astype(bf16), wh[_DIM:]) + bh).astype(bf16),
                wo) + bo
            h1 = dot(jax.nn.relu(
                dot(f1b, wh[:_DIM])
                + dot(mu1.astype(bf16), wh[_DIM:]) + bh).astype(bf16),
                wo) + bo
            h0_ref[pl.ds(base, _NBLK), :] = h0
            h1_ref[pl.ds(base, _NBLK), :] = h1
        f0_ref[...] = f0_ref[...] + h0_ref[...]
        f1_ref[...] = f1_ref[...] + h1_ref[...]

    out0_ref[0] = f0_ref[...]
    out1_ref[0] = f1_ref[...]


bcast = lambda shape: pl.BlockSpec(shape, lambda g: (0,) * len(shape))
per_g = lambda shape: pl.BlockSpec(shape, lambda g: (g,) + (0,) * (len(shape) - 1))


def _kernel_tc(input_xyz, coord_xyz, input_f, coord_f, Wf, bf, Ws, bs, gamma,
               beta, Wh, bh, Wo, bo):
    f32 = jnp.float32
    xyz = jnp.concatenate([input_xyz, coord_xyz], axis=0)
    xyzt = jnp.pad(jnp.transpose(xyz, (0, 2, 1)),
                   ((0, 0), (0, 5), (0, 0)))
    x_s = jnp.concatenate([input_f, coord_f], axis=0)

    agg, sums = pl.pallas_call(
        _knn_cgconv_body,
        grid=(2 * _B,),
        in_specs=[
            per_g((1, _NP, 3)),
            per_g((1, 8, _NP)),
            per_g((1, _NP, _DIM)),
            bcast((2 * _DIM, _DIM)),
            bcast((1, _DIM)),
            bcast((2 * _DIM, _DIM)),
            bcast((1, _DIM)),
        ],
        out_specs=[
            per_g((1, _NP, _DIM)),
            pl.BlockSpec((1, 2, _DIM), lambda g: (g // _B, 0, 0)),
        ],
        out_shape=[
            jax.ShapeDtypeStruct((2 * _B, _NP, _DIM), f32),
            jax.ShapeDtypeStruct((2, 2, _DIM), f32),
        ],
        scratch_shapes=[
            pltpu.VMEM((_NP, _DIM), f32),
            pltpu.VMEM((_NP, _DIM), f32),
            pltpu.VMEM((_NP, 2 * _DIM), jnp.bfloat16),
            pltpu.VMEM((_NP, 2 * _DIM), jnp.bfloat16),
        ],
    )(xyz, xyzt, x_s, Wf, bf.reshape(1, _DIM), Ws, bs.reshape(1, _DIM))
    return _run_cross_prop(agg, sums, input_f, coord_f, gamma, beta,
                           Wh, bh, Wo, bo)


def _run_cross_prop(agg, sums, input_f, coord_f, gamma, beta, Wh, bh, Wo, bo):
    f32 = jnp.float32
    out0, out1 = pl.pallas_call(
        _cross_prop_body,
        grid=(_B,),
        in_specs=[
            per_g((1, _NP, _DIM)),
            per_g((1, _NP, _DIM)),
            pl.BlockSpec((1, _NP, _DIM), lambda b: (b, 0, 0)),
            pl.BlockSpec((1, _NP, _DIM), lambda b: (b + _B, 0, 0)),
            bcast((2, 2, _DIM)),
            bcast((1, _DIM)),
            bcast((1, _DIM)),
            bcast((_NPROP, 2 * _DIM, _HID)),
            bcast((_NPROP, 1, _HID)),
            bcast((_NPROP, _HID, _DIM)),
            bcast((_NPROP, 1, _DIM)),
        ],
        out_specs=[
            per_g((1, _NP, _DIM)),
            per_g((1, _NP, _DIM)),
        ],
        out_shape=[
            jax.ShapeDtypeStruct((_B, _NP, _DIM), f32),
            jax.ShapeDtypeStruct((_B, _NP, _DIM), f32),
        ],
        scratch_shapes=[
            pltpu.VMEM((_NP, _DIM), f32),
            pltpu.VMEM((_NP, _DIM), f32),
            pltpu.VMEM((_NP, _DIM), f32),
            pltpu.VMEM((_NP, _DIM), f32),
        ],
    )(input_f, coord_f, agg, agg, sums, gamma.reshape(1, _DIM),
      beta.reshape(1, _DIM), Wh, bh.reshape(_NPROP, 1, _HID), Wo,
      bo.reshape(_NPROP, 1, _DIM))

    return (out0.reshape(-1, _DIM), out1.reshape(-1, _DIM))


def _knn_pqrs(xyz, x, Wf, bf, Ws, bs):
    f32 = jnp.float32
    xyzt = jnp.pad(jnp.transpose(xyz, (0, 2, 1)), ((0, 0), (0, 5), (0, 0)))
    return pl.pallas_call(
        _knn_pqrs_body,
        grid=(_B,),
        in_specs=[
            per_g((1, _NP, 3)),
            per_g((1, 8, _NP)),
            per_g((1, _NP, _DIM)),
            bcast((2 * _DIM, _DIM)),
            bcast((1, _DIM)),
            bcast((2 * _DIM, _DIM)),
            bcast((1, _DIM)),
        ],
        out_specs=[
            per_g((1, _NP, _DIM)),
            per_g((1, _NP, _DIM)),
            per_g((1, _NP, 2 * _DIM)),
            per_g((1, _NP, _K)),
        ],
        out_shape=[
            jax.ShapeDtypeStruct((_B, _NP, _DIM), f32),
            jax.ShapeDtypeStruct((_B, _NP, _DIM), f32),
            jax.ShapeDtypeStruct((_B, _NP, 2 * _DIM), f32),
            jax.ShapeDtypeStruct((_B, _NP, _K), jnp.int32),
        ],
    )(xyz, xyzt, x, Wf, bf.reshape(1, _DIM), Ws, bs.reshape(1, _DIM))


def _sc_gather(qs, idx):
    f32 = jnp.float32
    n_nodes = _B * _NP
    n_edges = n_nodes * _K
    gidx = idx + (jnp.arange(_B, dtype=jnp.int32) * _NP)[:, None, None]
    eidx = jnp.transpose(gidx.reshape(n_nodes, _K)).reshape(-1)
    mesh = plsc.VectorSubcoreMesh(core_axis_name="c", subcore_axis_name="s")
    return pl.kernel(
        _sc_gather_body,
        mesh=mesh,
        out_type=jax.ShapeDtypeStruct((n_edges, 2 * _DIM), f32),
        scratch_types=[
            pltpu.VMEM((_ECH,), jnp.int32),
            pltpu.VMEM((_ECH, 2 * _DIM), f32),
            pltpu.SemaphoreType.DMA,
        ],
    )(qs.reshape(n_nodes, 2 * _DIM), eidx)


def _msg_max(gathered, p, r):
    f32 = jnp.float32
    n_nodes = _B * _NP
    return pl.pallas_call(
        _msg_max_body,
        grid=(_B, _NP // _NBLK),
        in_specs=[
            pl.BlockSpec((_K, _NBLK, 2 * _DIM),
                         lambda g, nb: (0, g * (_NP // _NBLK) + nb, 0)),
            pl.BlockSpec((_NBLK, _DIM),
                         lambda g, nb: (g * (_NP // _NBLK) + nb, 0)),
            pl.BlockSpec((_NBLK, _DIM),
                         lambda g, nb: (g * (_NP // _NBLK) + nb, 0)),
        ],
        out_specs=[
            pl.BlockSpec((_NBLK, _DIM),
                         lambda g, nb: (g * (_NP // _NBLK) + nb, 0)),
            pl.BlockSpec((1, 2, _DIM), lambda g, nb: (0, 0, 0)),
        ],
        out_shape=[
            jax.ShapeDtypeStruct((n_nodes, _DIM), f32),
            jax.ShapeDtypeStruct((1, 2, _DIM), f32),
        ],
    )(gathered.reshape(_K, n_nodes, 2 * _DIM),
      p.reshape(n_nodes, _DIM), r.reshape(n_nodes, _DIM))


def _kernel_sc(input_xyz, coord_xyz, input_f, coord_f, Wf, bf, Ws, bs, gamma,
               beta, Wh, bh, Wo, bo):
    p0, r0, qs0, idx0 = _knn_pqrs(input_xyz, input_f, Wf, bf, Ws, bs)
    g0 = _sc_gather(qs0, idx0)
    p1, r1, qs1, idx1 = _knn_pqrs(coord_xyz, coord_f, Wf, bf, Ws, bs)
    g1 = _sc_gather(qs1, idx1)
    agg0, sums0 = _msg_max(g0, p0, r0)
    agg1, sums1 = _msg_max(g1, p1, r1)
    agg = jnp.concatenate([agg0.reshape(_B, _NP, _DIM),
                           agg1.reshape(_B, _NP, _DIM)], axis=0)
    sums = jnp.concatenate([sums0, sums1], axis=0)
    return _run_cross_prop(agg, sums, input_f, coord_f, gamma, beta,
                           Wh, bh, Wo, bo)


def kernel(input_xyz, coord_xyz, input_f, coord_f, Wf, bf, Ws, bs, gamma,
           beta, Wh, bh, Wo, bo):
    return _kernel_sc(input_xyz, coord_xyz, input_f, coord_f, Wf, bf, Ws,
                      bs, gamma, beta, Wh, bh, Wo, bo)


# speedup vs baseline: 1.0571x; 1.0345x over previous
import functools

import jax
import jax.numpy as jnp
from jax import lax
from jax.experimental import pallas as pl
from jax.experimental.pallas import tpu as pltpu
from jax.experimental.pallas import tpu_sc as plsc

_B, _NP, _DIM, _K = 4, 1024, 128, 16
_HID, _NPROP = 64, 5
_NBLK = 256
_HI = jax.lax.Precision.HIGHEST


def _sigmoid(u):
    return 1.0 / (1.0 + jnp.exp(-u))


def _softplus(v):
    return jnp.maximum(v, 0.0) + jnp.log1p(jnp.exp(-jnp.abs(v)))


def _knn_cgconv_body(xyz_ref, xyzt_ref, x_ref, wf_ref, bf_ref, ws_ref, bs_ref,
                     agg_ref, sums_ref, p_ref, r_ref, qsh_ref, qsl_ref):
    g = pl.program_id(0)
    b_in_graph = jax.lax.rem(g, _B)

    x = x_ref[0].astype(jnp.bfloat16)
    wf = wf_ref[...].astype(jnp.bfloat16)
    ws = ws_ref[...].astype(jnp.bfloat16)
    dot = functools.partial(jnp.dot, preferred_element_type=jnp.float32)
    p_ref[...] = dot(x, wf[:_DIM]) + bf_ref[...]
    r_ref[...] = dot(x, ws[:_DIM]) + bs_ref[...]
    q = dot(x, wf[_DIM:])
    s = dot(x, ws[_DIM:])
    qs = jnp.concatenate([q, s], axis=1)
    hi = qs.astype(jnp.bfloat16)
    qsh_ref[...] = hi
    qsl_ref[...] = (qs - hi.astype(jnp.float32)).astype(jnp.bfloat16)

    total_sum = jnp.zeros((1, _DIM), jnp.float32)
    total_sq = jnp.zeros((1, _DIM), jnp.float32)

    for nb in range(_NP // _NBLK):
        base = nb * _NBLK
        lane = jax.lax.broadcasted_iota(jnp.int32, (_NBLK, _NP), 1)
        rows = base + jax.lax.broadcasted_iota(jnp.int32, (_NBLK, _NP), 0)
        d2 = jnp.zeros((_NBLK, _NP), jnp.float32)
        for c in range(3):
            col = xyz_ref[0, pl.ds(base, _NBLK), c:c + 1]
            row = xyzt_ref[0, c:c + 1, :]
            t = col - row
            d2 = d2 + t * t
        d2 = d2 + jnp.where(lane == rows, 1e10, 0.0)

        p_blk = p_ref[pl.ds(base, _NBLK), :]
        r_blk = r_ref[pl.ds(base, _NBLK), :]
        qsh = qsh_ref[...]
        qsl = qsl_ref[...]
        acc = None
        for k in range(_K):
            m = jnp.min(d2, axis=1, keepdims=True)
            cur = jnp.min(jnp.where(d2 == m, lane, jnp.int32(2**30)),
                          axis=1, keepdims=True)
            hit = lane == cur
            d2 = jnp.where(hit, 3e30, d2)
            oh = jnp.where(hit, 1.0, 0.0).astype(jnp.bfloat16)
            gh = jax.lax.dot_general(oh, qsh, (((1,), (0,)), ((), ())),
                                     preferred_element_type=jnp.float32)
            gl = jax.lax.dot_general(oh, qsl, (((1,), (0,)), ((), ())),
                                     preferred_element_type=jnp.float32)
            gat = gh + gl
            u = p_blk + gat[:, :_DIM]
            v = r_blk + gat[:, _DIM:]
            msg = _sigmoid(u) * _softplus(v)
            acc = msg if acc is None else jnp.maximum(acc, msg)
        agg_ref[0, pl.ds(base, _NBLK), :] = acc
        total_sum = total_sum + jnp.sum(acc, axis=0, keepdims=True)
        total_sq = total_sq + jnp.sum(acc * acc, axis=0, keepdims=True)

    local = jnp.concatenate([total_sum[:, None, :], total_sq[:, None, :]],
                            axis=1)

    @pl.when(b_in_graph == 0)
    def _():
        sums_ref[...] = local

    @pl.when(b_in_graph != 0)
    def _():
        sums_ref[...] = sums_ref[...] + local


def _knn_pqrs_body(xyz_ref, xyzt_ref, x_ref, wf_ref, bf_ref, ws_ref, bs_ref,
                   p_ref, r_ref, qs_ref, idx_ref):
    x = x_ref[0].astype(jnp.bfloat16)
    wf = wf_ref[...].astype(jnp.bfloat16)
    ws = ws_ref[...].astype(jnp.bfloat16)
    dot = functools.partial(jnp.dot, preferred_element_type=jnp.float32)
    p_ref[0] = dot(x, wf[:_DIM]) + bf_ref[...]
    r_ref[0] = dot(x, ws[:_DIM]) + bs_ref[...]
    qs_ref[0, :, :_DIM] = dot(x, wf[_DIM:])
    qs_ref[0, :, _DIM:] = dot(x, ws[_DIM:])

    for nb in range(_NP // _NBLK):
        base = nb * _NBLK
        lane = jax.lax.broadcasted_iota(jnp.int32, (_NBLK, _NP), 1)
        rows = base + jax.lax.broadcasted_iota(jnp.int32, (_NBLK, _NP), 0)
        d2 = jnp.zeros((_NBLK, _NP), jnp.float32)
        for c in range(3):
            col = xyz_ref[0, pl.ds(base, _NBLK), c:c + 1]
            row = xyzt_ref[0, c:c + 1, :]
            t = col - row
            d2 = d2 + t * t
        d2 = d2 + jnp.where(lane == rows, 1e10, 0.0)
        for k in range(_K):
            m = jnp.min(d2, axis=1, keepdims=True)
            cur = jnp.min(jnp.where(d2 == m, lane, jnp.int32(2**30)),
                          axis=1, keepdims=True)
            d2 = jnp.where(lane == cur, 3e30, d2)
            idx_ref[0, pl.ds(base, _NBLK), k:k + 1] = cur


_NW = 32
_ECH = 128


def _sc_gather_body(table_ref, idx_ref, out_ref, idx_v0, idx_v1,
                    rows_v0, rows_v1, sem0, sem1):
    wid = lax.axis_index("s") * 2 + lax.axis_index("c")
    per_w = (_B * _NP * _K) // _NW
    base = wid * per_w

    @pl.loop(0, per_w // _ECH, step=2)
    def _(j):
        off0 = base + j * _ECH
        off1 = off0 + _ECH
        pltpu.sync_copy(idx_ref.at[pl.ds(off0, _ECH)], idx_v0)
        cp0 = pltpu.async_copy(table_ref.at[idx_v0], rows_v0, sem0)
        pltpu.sync_copy(idx_ref.at[pl.ds(off1, _ECH)], idx_v1)
        cp1 = pltpu.async_copy(table_ref.at[idx_v1], rows_v1, sem1)
        cp0.wait()
        pltpu.sync_copy(rows_v0, out_ref.at[pl.ds(off0, _ECH)])
        cp1.wait()
        pltpu.sync_copy(rows_v1, out_ref.at[pl.ds(off1, _ECH)])


def _msg_max_body(g_ref, p_ref, r_ref, agg_ref, sums_ref):
    g = pl.program_id(0)
    nb = pl.program_id(1)
    p = p_ref[...]
    r = r_ref[...]
    acc = None
    for k in range(_K):
        gat = g_ref[k]
        u = p + gat[:, :_DIM]
        v = r + gat[:, _DIM:]
        msg = _sigmoid(u) * _softplus(v)
        acc = msg if acc is None else jnp.maximum(acc, msg)
    agg_ref[...] = acc
    bsum = jnp.sum(acc, axis=0, keepdims=True)
    bsq = jnp.sum(acc * acc, axis=0, keepdims=True)
    local = jnp.concatenate([bsum[:, None, :], bsq[:, None, :]], axis=1)

    first = jnp.logical_and(lax.rem(g, _B) == 0, nb == 0)

    @pl.when(first)
    def _():
        sums_ref[...] = local

    @pl.when(jnp.logical_not(first))
    def _():
        sums_ref[...] = sums_ref[...] + local


def _cross_prop_body(x0_ref, x1_ref, agg0_ref, agg1_ref, sums_ref,
                     gamma_ref, beta_ref, wh_ref, bh_ref, wo_ref, bo_ref,
                     out0_ref, out1_ref, f0_ref, f1_ref, h0_ref, h1_ref):
    gamma = gamma_ref[...]
    beta = beta_ref[...]
    n_nodes = jnp.float32(_B * _NP)

    for gi, (x_ref, agg_ref, f_ref) in enumerate(
            ((x0_ref, agg0_ref, f0_ref), (x1_ref, agg1_ref, f1_ref))):
        ssum = sums_ref[gi, 0:1, :]
        ssq = sums_ref[gi, 1:2, :]
        mu = ssum / n_nodes
        var = ssq / n_nodes - mu * mu
        inv = jax.lax.rsqrt(var + 1e-5)
        f_ref[...] = x_ref[0] + (agg_ref[0] - mu) * inv * gamma + beta

    def rowsoftmax(t):
        m = jnp.max(t, axis=1, keepdims=True)
        e = jnp.exp(t - m)
        return e / jnp.sum(e, axis=1, keepdims=True)

    bf16 = jnp.bfloat16
    dot = functools.partial(jnp.dot, preferred_element_type=jnp.float32)
    for l in range(_NPROP):
        wh = wh_ref[l].astype(bf16)
        bh = bh_ref[l]
        wo = wo_ref[l].astype(bf16)
        bo = bo_ref[l]
        f0 = f0_ref[...].astype(bf16)
        f1 = f1_ref[...].astype(bf16)
        for nb in range(_NP // _NBLK):
            base = nb * _NBLK
            f0b = f0[base:base + _NBLK, :]
            f1b = f1[base:base + _NBLK, :]
            s = jax.lax.dot_general(f0b, f1, (((1,), (1,)), ((), ())),
                                    preferred_element_type=jnp.float32)
            st = jax.lax.dot_general(f1b, f0, (((1,), (1,)), ((), ())),
                                     preferred_element_type=jnp.float32)
            a0 = rowsoftmax(s).astype(bf16)
            a1t = rowsoftmax(st).astype(bf16)
            att0 = dot(a0, f1)
            att1 = dot(a1t, f0)
            mu0 = f0_ref[pl.ds(base, _NBLK), :] - att0
            mu1 = f1_ref[pl.ds(base, _NBLK), :] - att1
            h0 = dot(jax.nn.relu(
                dot(f0b, wh[:_DIM])
                + dot(mu0.astype(bf16), wh[_DIM:]) + bh).astype(bf16),
                wo) + bo
            h1 = dot(jax.nn.relu(
                dot(f1b, wh[:_DIM])
                + dot(mu1.astype(bf16), wh[_DIM:]) + bh).astype(bf16),
                wo) + bo
            h0_ref[pl.ds(base, _NBLK), :] = h0
            h1_ref[pl.ds(base, _NBLK), :] = h1
        f0_ref[...] = f0_ref[...] + h0_ref[...]
        f1_ref[...] = f1_ref[...] + h1_ref[...]

    out0_ref[0] = f0_ref[...]
    out1_ref[0] = f1_ref[...]


bcast = lambda shape: pl.BlockSpec(shape, lambda g: (0,) * len(shape))
per_g = lambda shape: pl.BlockSpec(shape, lambda g: (g,) + (0,) * (len(shape) - 1))


def _kernel_tc(input_xyz, coord_xyz, input_f, coord_f, Wf, bf, Ws, bs, gamma,
               beta, Wh, bh, Wo, bo):
    f32 = jnp.float32
    xyz = jnp.concatenate([input_xyz, coord_xyz], axis=0)
    xyzt = jnp.pad(jnp.transpose(xyz, (0, 2, 1)),
                   ((0, 0), (0, 5), (0, 0)))
    x_s = jnp.concatenate([input_f, coord_f], axis=0)

    agg, sums = pl.pallas_call(
        _knn_cgconv_body,
        grid=(2 * _B,),
        in_specs=[
            per_g((1, _NP, 3)),
            per_g((1, 8, _NP)),
            per_g((1, _NP, _DIM)),
            bcast((2 * _DIM, _DIM)),
            bcast((1, _DIM)),
            bcast((2 * _DIM, _DIM)),
            bcast((1, _DIM)),
        ],
        out_specs=[
            per_g((1, _NP, _DIM)),
            pl.BlockSpec((1, 2, _DIM), lambda g: (g // _B, 0, 0)),
        ],
        out_shape=[
            jax.ShapeDtypeStruct((2 * _B, _NP, _DIM), f32),
            jax.ShapeDtypeStruct((2, 2, _DIM), f32),
        ],
        scratch_shapes=[
            pltpu.VMEM((_NP, _DIM), f32),
            pltpu.VMEM((_NP, _DIM), f32),
            pltpu.VMEM((_NP, 2 * _DIM), jnp.bfloat16),
            pltpu.VMEM((_NP, 2 * _DIM), jnp.bfloat16),
        ],
    )(xyz, xyzt, x_s, Wf, bf.reshape(1, _DIM), Ws, bs.reshape(1, _DIM))
    return _run_cross_prop(agg, sums, input_f, coord_f, gamma, beta,
                           Wh, bh, Wo, bo)


def _run_cross_prop(agg, sums, input_f, coord_f, gamma, beta, Wh, bh, Wo, bo):
    f32 = jnp.float32
    out0, out1 = pl.pallas_call(
        _cross_prop_body,
        grid=(_B,),
        in_specs=[
            per_g((1, _NP, _DIM)),
            per_g((1, _NP, _DIM)),
            pl.BlockSpec((1, _NP, _DIM), lambda b: (b, 0, 0)),
            pl.BlockSpec((1, _NP, _DIM), lambda b: (b + _B, 0, 0)),
            bcast((2, 2, _DIM)),
            bcast((1, _DIM)),
            bcast((1, _DIM)),
            bcast((_NPROP, 2 * _DIM, _HID)),
            bcast((_NPROP, 1, _HID)),
            bcast((_NPROP, _HID, _DIM)),
            bcast((_NPROP, 1, _DIM)),
        ],
        out_specs=[
            per_g((1, _NP, _DIM)),
            per_g((1, _NP, _DIM)),
        ],
        out_shape=[
            jax.ShapeDtypeStruct((_B, _NP, _DIM), f32),
            jax.ShapeDtypeStruct((_B, _NP, _DIM), f32),
        ],
        scratch_shapes=[
            pltpu.VMEM((_NP, _DIM), f32),
            pltpu.VMEM((_NP, _DIM), f32),
            pltpu.VMEM((_NP, _DIM), f32),
            pltpu.VMEM((_NP, _DIM), f32),
        ],
    )(input_f, coord_f, agg, agg, sums, gamma.reshape(1, _DIM),
      beta.reshape(1, _DIM), Wh, bh.reshape(_NPROP, 1, _HID), Wo,
      bo.reshape(_NPROP, 1, _DIM))

    return (out0.reshape(-1, _DIM), out1.reshape(-1, _DIM))


def _knn_pqrs(xyz, x, Wf, bf, Ws, bs):
    f32 = jnp.float32
    xyzt = jnp.pad(jnp.transpose(xyz, (0, 2, 1)), ((0, 0), (0, 5), (0, 0)))
    return pl.pallas_call(
        _knn_pqrs_body,
        grid=(_B,),
        in_specs=[
            per_g((1, _NP, 3)),
            per_g((1, 8, _NP)),
            per_g((1, _NP, _DIM)),
            bcast((2 * _DIM, _DIM)),
            bcast((1, _DIM)),
            bcast((2 * _DIM, _DIM)),
            bcast((1, _DIM)),
        ],
        out_specs=[
            per_g((1, _NP, _DIM)),
            per_g((1, _NP, _DIM)),
            per_g((1, _NP, 2 * _DIM)),
            per_g((1, _NP, _K)),
        ],
        out_shape=[
            jax.ShapeDtypeStruct((_B, _NP, _DIM), f32),
            jax.ShapeDtypeStruct((_B, _NP, _DIM), f32),
            jax.ShapeDtypeStruct((_B, _NP, 2 * _DIM), f32),
            jax.ShapeDtypeStruct((_B, _NP, _K), jnp.int32),
        ],
    )(xyz, xyzt, x, Wf, bf.reshape(1, _DIM), Ws, bs.reshape(1, _DIM))


def _sc_gather(qs, idx):
    f32 = jnp.float32
    n_nodes = _B * _NP
    n_edges = n_nodes * _K
    gidx = idx + (jnp.arange(_B, dtype=jnp.int32) * _NP)[:, None, None]
    eidx = jnp.transpose(gidx.reshape(n_nodes, _K)).reshape(-1)
    mesh = plsc.VectorSubcoreMesh(core_axis_name="c", subcore_axis_name="s")
    return pl.kernel(
        _sc_gather_body,
        mesh=mesh,
        out_type=jax.ShapeDtypeStruct((n_edges, 2 * _DIM), f32),
        scratch_types=[
            pltpu.VMEM((_ECH,), jnp.int32),
            pltpu.VMEM((_ECH,), jnp.int32),
            pltpu.VMEM((_ECH, 2 * _DIM), f32),
            pltpu.VMEM((_ECH, 2 * _DIM), f32),
            pltpu.SemaphoreType.DMA,
            pltpu.SemaphoreType.DMA,
        ],
    )(qs.reshape(n_nodes, 2 * _DIM), eidx)


def _msg_max(gathered, p, r):
    f32 = jnp.float32
    n_nodes = _B * _NP
    return pl.pallas_call(
        _msg_max_body,
        grid=(_B, _NP // _NBLK),
        in_specs=[
            pl.BlockSpec((_K, _NBLK, 2 * _DIM),
                         lambda g, nb: (0, g * (_NP // _NBLK) + nb, 0)),
            pl.BlockSpec((_NBLK, _DIM),
                         lambda g, nb: (g * (_NP // _NBLK) + nb, 0)),
            pl.BlockSpec((_NBLK, _DIM),
                         lambda g, nb: (g * (_NP // _NBLK) + nb, 0)),
        ],
        out_specs=[
            pl.BlockSpec((_NBLK, _DIM),
                         lambda g, nb: (g * (_NP // _NBLK) + nb, 0)),
            pl.BlockSpec((1, 2, _DIM), lambda g, nb: (0, 0, 0)),
        ],
        out_shape=[
            jax.ShapeDtypeStruct((n_nodes, _DIM), f32),
            jax.ShapeDtypeStruct((1, 2, _DIM), f32),
        ],
    )(gathered.reshape(_K, n_nodes, 2 * _DIM),
      p.reshape(n_nodes, _DIM), r.reshape(n_nodes, _DIM))


def _kernel_sc(input_xyz, coord_xyz, input_f, coord_f, Wf, bf, Ws, bs, gamma,
               beta, Wh, bh, Wo, bo):
    p0, r0, qs0, idx0 = _knn_pqrs(input_xyz, input_f, Wf, bf, Ws, bs)
    g0 = _sc_gather(qs0, idx0)
    p1, r1, qs1, idx1 = _knn_pqrs(coord_xyz, coord_f, Wf, bf, Ws, bs)
    g1 = _sc_gather(qs1, idx1)
    agg0, sums0 = _msg_max(g0, p0, r0)
    agg1, sums1 = _msg_max(g1, p1, r1)
    agg = jnp.concatenate([agg0.reshape(_B, _NP, _DIM),
                           agg1.reshape(_B, _NP, _DIM)], axis=0)
    sums = jnp.concatenate([sums0, sums1], axis=0)
    return _run_cross_prop(agg, sums, input_f, coord_f, gamma, beta,
                           Wh, bh, Wo, bo)


def kernel(input_xyz, coord_xyz, input_f, coord_f, Wf, bf, Ws, bs, gamma,
           beta, Wh, bh, Wo, bo):
    return _kernel_sc(input_xyz, coord_xyz, input_f, coord_f, Wf, bf, Ws,
                      bs, gamma, beta, Wh, bh, Wo, bo)
